# baseline (speedup 1.0000x reference)
import functools

import jax
import jax.numpy as jnp
from jax import lax
from jax.experimental import pallas as pl
from jax.experimental.pallas import tpu as pltpu
from jax.experimental.pallas import tpu_sc as plsc

_D = 128
_NBUF = 4


@functools.lru_cache(maxsize=None)
def _build(n_seq, seq_len):
    info = plsc.get_sparse_core_info()
    NC, NS = info.num_cores, info.num_subcores
    NW = NC * NS
    assert n_seq % NW == 0
    SPW = n_seq // NW
    assert SPW % _NBUF == 0
    NOUT = SPW // _NBUF

    mesh = plsc.VectorSubcoreMesh(core_axis_name="c", subcore_axis_name="s")

    @functools.partial(
        pl.kernel,
        mesh=mesh,
        out_type=jax.ShapeDtypeStruct((n_seq, seq_len, _D), jnp.float32),
        scratch_types=[
            pltpu.VMEM((SPW, seq_len), jnp.int32),
            pltpu.VMEM((_NBUF, seq_len, _D), jnp.float32),
            pltpu.SemaphoreType.DMA((_NBUF,)),
            pltpu.SemaphoreType.DMA((_NBUF,)),
        ],
    )
    def emb(table_hbm, ids_hbm, out_hbm, idx_v, bufs, gsem, osem):
        wid = lax.axis_index("s") * NC + lax.axis_index("c")
        seq_base = wid * SPW
        pltpu.sync_copy(ids_hbm.at[pl.ds(seq_base, SPW)], idx_v)
        for b in range(_NBUF):
            pltpu.async_copy(table_hbm.at[idx_v.at[b]], bufs.at[b], gsem.at[b])

        def outer(g, carry):
            for b in range(_NBUF):
                j = g * _NBUF + b
                pltpu.make_async_copy(
                    table_hbm.at[idx_v.at[j]], bufs.at[b], gsem.at[b]
                ).wait()
                pltpu.async_copy(bufs.at[b], out_hbm.at[seq_base + j], osem.at[b])

                @pl.when(g < NOUT - 1)
                def _():
                    pltpu.make_async_copy(
                        bufs.at[b], out_hbm.at[seq_base + j], osem.at[b]
                    ).wait()
                    pltpu.async_copy(
                        table_hbm.at[idx_v.at[j + _NBUF]], bufs.at[b], gsem.at[b]
                    )
            return carry

        lax.fori_loop(0, NOUT, outer, 0)
        for b in range(_NBUF):
            j = (NOUT - 1) * _NBUF + b
            pltpu.make_async_copy(
                bufs.at[b], out_hbm.at[seq_base + j], osem.at[b]
            ).wait()

    return emb


def kernel(token_ids, embedding):
    n_seq, seq_len = token_ids.shape
    emb = _build(n_seq, seq_len)
    return emb(embedding, token_ids.astype(jnp.int32))


# speedup vs baseline: 1.8036x; 1.8036x over previous
import functools

import jax
import jax.numpy as jnp
from jax import lax
from jax.experimental import pallas as pl
from jax.experimental.pallas import tpu as pltpu
from jax.experimental.pallas import tpu_sc as plsc

_D = 128
_NBUF = 5


@functools.lru_cache(maxsize=None)
def _build(n_seq, seq_len):
    info = plsc.get_sparse_core_info()
    NC, NS = info.num_cores, info.num_subcores
    NW = NC * NS
    assert n_seq % NW == 0
    SPW = n_seq // NW
    assert seq_len % _NBUF == 0
    NOUT = seq_len // _NBUF

    mesh = plsc.VectorSubcoreMesh(core_axis_name="c", subcore_axis_name="s")

    @functools.partial(
        pl.kernel,
        mesh=mesh,
        out_type=jax.ShapeDtypeStruct((seq_len, n_seq, _D), jnp.float32),
        scratch_types=[
            pltpu.VMEM((seq_len, SPW), jnp.int32),
            pltpu.VMEM((_NBUF, SPW, _D), jnp.float32),
            pltpu.SemaphoreType.DMA((_NBUF,)),
            pltpu.SemaphoreType.DMA((_NBUF,)),
        ],
    )
    def emb(table_hbm, ids_hbm, out_hbm, idx_v, bufs, gsem, osem):
        wid = lax.axis_index("s") * NC + lax.axis_index("c")
        base = wid * SPW
        pltpu.sync_copy(ids_hbm.at[:, pl.ds(base, SPW)], idx_v)
        for b in range(_NBUF):
            pltpu.async_copy(table_hbm.at[idx_v.at[b]], bufs.at[b], gsem.at[b])

        def outer(g, carry):
            for b in range(_NBUF):
                j = g * _NBUF + b
                dst = out_hbm.at[j].at[pl.ds(base, SPW)]
                pltpu.make_async_copy(
                    table_hbm.at[idx_v.at[j]], bufs.at[b], gsem.at[b]
                ).wait()
                pltpu.async_copy(bufs.at[b], dst, osem.at[b])

                @pl.when(g < NOUT - 1)
                def _():
                    pltpu.make_async_copy(bufs.at[b], dst, osem.at[b]).wait()
                    pltpu.async_copy(
                        table_hbm.at[idx_v.at[j + _NBUF]], bufs.at[b], gsem.at[b]
                    )
            return carry

        lax.fori_loop(0, NOUT, outer, 0)
        for b in range(_NBUF):
            j = (NOUT - 1) * _NBUF + b
            pltpu.make_async_copy(
                bufs.at[b], out_hbm.at[j].at[pl.ds(base, SPW)], osem.at[b]
            ).wait()

    return emb


def kernel(token_ids, embedding):
    n_seq, seq_len = token_ids.shape
    emb = _build(n_seq, seq_len)
    ids_t = jnp.transpose(token_ids).astype(jnp.int32)
    out_t = emb(embedding, ids_t)
    return jnp.transpose(out_t, (1, 0, 2))


# speedup vs baseline: 1.8170x; 1.0074x over previous
import functools

import jax
import jax.numpy as jnp
from jax import lax
from jax.experimental import pallas as pl
from jax.experimental.pallas import tpu as pltpu
from jax.experimental.pallas import tpu_sc as plsc

_D = 128
_NBUF = 10
_HALF = 2


@functools.lru_cache(maxsize=None)
def _build(n_seq, seq_len):
    info = plsc.get_sparse_core_info()
    NC, NS = info.num_cores, info.num_subcores
    NW = NC * NS
    assert n_seq % NW == 0
    SPW = n_seq // NW
    CH = SPW // _HALF
    NCH = seq_len * _HALF
    assert NCH % _NBUF == 0
    NOUT = NCH // _NBUF

    mesh = plsc.VectorSubcoreMesh(core_axis_name="c", subcore_axis_name="s")

    @functools.partial(
        pl.kernel,
        mesh=mesh,
        out_type=jax.ShapeDtypeStruct((seq_len, n_seq, _D), jnp.float32),
        scratch_types=[
            pltpu.VMEM((seq_len, SPW), jnp.int32),
            pltpu.VMEM((_NBUF, CH, _D), jnp.float32),
            pltpu.SemaphoreType.DMA((_NBUF,)),
            pltpu.SemaphoreType.DMA((_NBUF,)),
        ],
    )
    def emb(table_hbm, ids_hbm, out_hbm, idx_v, bufs, gsem, osem):
        wid = lax.axis_index("s") * NC + lax.axis_index("c")
        base = wid * SPW
        pltpu.sync_copy(ids_hbm.at[:, pl.ds(base, SPW)], idx_v)

        def idx_of(j):
            return idx_v.at[j // _HALF].at[pl.ds((j % _HALF) * CH, CH)]

        def dst_of(j):
            return out_hbm.at[j // _HALF].at[pl.ds(base + (j % _HALF) * CH, CH)]

        for b in range(_NBUF):
            pltpu.async_copy(table_hbm.at[idx_of(b)], bufs.at[b], gsem.at[b])

        def outer(g, carry):
            for b in range(_NBUF):
                j = g * _NBUF + b
                pltpu.make_async_copy(
                    table_hbm.at[idx_of(j)], bufs.at[b], gsem.at[b]
                ).wait()
                pltpu.async_copy(bufs.at[b], dst_of(j), osem.at[b])

                @pl.when(g < NOUT - 1)
                def _():
                    pltpu.make_async_copy(bufs.at[b], dst_of(j), osem.at[b]).wait()
                    pltpu.async_copy(
                        table_hbm.at[idx_of(j + _NBUF)], bufs.at[b], gsem.at[b]
                    )
            return carry

        lax.fori_loop(0, NOUT, outer, 0)
        for b in range(_NBUF):
            j = (NOUT - 1) * _NBUF + b
            pltpu.make_async_copy(bufs.at[b], dst_of(j), osem.at[b]).wait()

    return emb


def kernel(token_ids, embedding):
    n_seq, seq_len = token_ids.shape
    emb = _build(n_seq, seq_len)
    ids_t = jnp.transpose(token_ids).astype(jnp.int32)
    out_t = emb(embedding, ids_t)
    return jnp.transpose(out_t, (1, 0, 2))
